# Initial kernel scaffold; baseline (speedup 1.0000x reference)
import jax
import jax.numpy as jnp
from jax.experimental import pallas as pl


def kernel(features, nodes, samples1, samples0, num_valid0, num_valid1, W1, W2):
    raise NotImplementedError("write your pallas kernel here")


# speedup vs baseline: 5.9369x; 5.9369x over previous
import functools

import jax
import jax.numpy as jnp
from jax import lax
from jax.experimental import pallas as pl
from jax.experimental.pallas import tpu as pltpu
from jax.experimental.pallas import tpu_sc as plsc

N = 100000
D = 128
B = 1024
S = 16

NC = 2
NS = 16
NW = NC * NS
L = 16

GPW = (B * S) // NW
CG = 16
NCH = GPW // CG
CROWS = CG * S
H1PW = (B * S) // NW
H2PW = B // NW
DCH = D // L


def _sc_body(features, samples0, samples1, nodes, nv0,
             agg0_o, h1_o, h2_o,
             idx_a, idx_b, rows_a, rows_b, nv_v, aggbuf, h2idx, h2rows,
             sem_a, sem_b):
    wid = lax.axis_index("s") * NC + lax.axis_index("c")

    b2 = wid * H2PW
    pltpu.sync_copy(nodes.at[pl.ds(b2, H2PW)], h2idx)
    pltpu.async_copy(features.at[h2idx], h2rows, sem_a).wait()
    pltpu.sync_copy(h2rows, h2_o.at[pl.ds(b2, H2PW)])

    def h1_chunk(c, carry):
        b1 = wid * H1PW + c * 128
        pltpu.sync_copy(samples1.at[pl.ds(b1, 128)], idx_a)
        pltpu.async_copy(features.at[idx_a], rows_a, sem_a).wait()
        pltpu.sync_copy(rows_a, h1_o.at[pl.ds(b1, 128)])
        return carry
    lax.fori_loop(0, H1PW // 128, h1_chunk, 0)

    gbase = wid * GPW
    pltpu.sync_copy(nv0.at[pl.ds(gbase, GPW)], nv_v)
    lanes = lax.iota(jnp.int32, L)

    def agg_chunk(c, carry):
        rbase = gbase * S + c * CROWS
        pltpu.sync_copy(samples0.at[pl.ds(rbase, 128)], idx_a)
        pltpu.sync_copy(samples0.at[pl.ds(rbase + 128, 128)], idx_b)
        cp_a = pltpu.async_copy(features.at[idx_a], rows_a, sem_a)
        cp_b = pltpu.async_copy(features.at[idx_b], rows_b, sem_b)
        cp_a.wait()
        cp_b.wait()
        nvvec = nv_v[pl.ds(c * CG, CG)]
        invvec = 1.0 / nvvec.astype(jnp.float32)
        for j in range(CG):
            rows = rows_a if j < 8 else rows_b
            lbase = (j % 8) * S
            nv_j = nvvec[j]

            def acc_row(i, accs):
                r = lbase + i
                return tuple(accs[d] + rows[r, pl.ds(d * L, L)]
                             for d in range(DCH))

            accs = lax.fori_loop(
                0, nv_j, acc_row,
                tuple(jnp.zeros((L,), jnp.float32) for _ in range(DCH)))
            inv = invvec[j]
            for d in range(DCH):
                aggbuf[j, pl.ds(d * L, L)] = accs[d] * inv
        pltpu.sync_copy(aggbuf, agg0_o.at[pl.ds(gbase + c * CG, CG)])
        return carry
    lax.fori_loop(0, NCH, agg_chunk, 0)


@jax.jit
def _sc_gather(features, samples0, samples1, nodes, nv0):
    mesh = plsc.VectorSubcoreMesh(core_axis_name="c", subcore_axis_name="s",
                                  num_cores=NC, num_subcores=NS)
    return pl.kernel(
        _sc_body,
        out_type=(
            jax.ShapeDtypeStruct((B * S, D), jnp.float32),
            jax.ShapeDtypeStruct((B * S, D), jnp.float32),
            jax.ShapeDtypeStruct((B, D), jnp.float32),
        ),
        mesh=mesh,
        scratch_types=[
            pltpu.VMEM((128,), jnp.int32),
            pltpu.VMEM((128,), jnp.int32),
            pltpu.VMEM((128, D), jnp.float32),
            pltpu.VMEM((128, D), jnp.float32),
            pltpu.VMEM((GPW,), jnp.int32),
            pltpu.VMEM((CG, D), jnp.float32),
            pltpu.VMEM((H2PW,), jnp.int32),
            pltpu.VMEM((H2PW, D), jnp.float32),
            pltpu.SemaphoreType.DMA,
            pltpu.SemaphoreType.DMA,
        ],
    )(features, samples0, samples1, nodes, nv0)


def _tc_body(h1, agg0, h2, w1, w2, m1, inv1, out):
    w1a = w1[:, :D]
    w1b = w1[:, D:]
    new1 = jnp.maximum(
        lax.dot_general(h1[:], w1a, (((1,), (1,)), ((), ())))
        + lax.dot_general(agg0[:], w1b, (((1,), (1,)), ((), ()))), 0.0)
    m = m1[:]
    agg2 = jnp.sum((new1 * m).reshape(B, S, D), axis=1) * inv1[:]
    agg1 = jnp.sum((h1[:] * m).reshape(B, S, D), axis=1) * inv1[:]
    new2 = jnp.maximum(
        lax.dot_general(h2[:], w1a, (((1,), (1,)), ((), ())))
        + lax.dot_general(agg1, w1b, (((1,), (1,)), ((), ()))), 0.0)
    out[:] = jnp.maximum(
        lax.dot_general(new2, w2[:, :D], (((1,), (1,)), ((), ())))
        + lax.dot_general(agg2, w2[:, D:], (((1,), (1,)), ((), ()))), 0.0)


@jax.jit
def _tc_combine(h1, agg0, h2, w1, w2, m1, inv1):
    return pl.pallas_call(
        _tc_body,
        out_shape=jax.ShapeDtypeStruct((B, D), jnp.float32),
    )(h1, agg0, h2, w1, w2, m1, inv1)


def kernel(features, nodes, samples1, samples0, num_valid0, num_valid1, W1, W2):
    nodes = nodes.astype(jnp.int32)
    samples1 = samples1.astype(jnp.int32)
    samples0 = samples0.astype(jnp.int32)
    nv0 = num_valid0.reshape(-1).astype(jnp.int32)
    agg0, h1, h2 = _sc_gather(features, samples0, samples1, nodes, nv0)
    m1 = (jnp.arange(S)[None, :] < num_valid1).astype(jnp.float32)
    m1 = m1.reshape(B * S, 1)
    inv1 = 1.0 / num_valid1.astype(jnp.float32)
    return _tc_combine(h1, agg0, h2, W1, W2, m1, inv1)


# speedup vs baseline: 9.1377x; 1.5391x over previous
import functools

import jax
import jax.numpy as jnp
from jax import lax
from jax.experimental import pallas as pl
from jax.experimental.pallas import tpu as pltpu
from jax.experimental.pallas import tpu_sc as plsc

N = 100000
D = 128
B = 1024
S = 16

NC = 2
NS = 16
NW = NC * NS
L = 16

GPW = (B * S) // NW
CG = 8
NCH = GPW // CG
CROWS = CG * S
H1PW = (B * S) // NW
H2PW = B // NW
DCH = D // L


def _sc_body(features, samples0, samples1, nodes, nv0,
             agg0_o, h1_o, h2_o,
             idx_all, rows0, rows1, nv_v, agg_v, h2idx, h2rows,
             sem0, sem1):
    wid = lax.axis_index("s") * NC + lax.axis_index("c")
    rows_bufs = (rows0, rows1)
    sems = (sem0, sem1)

    b2 = wid * H2PW
    pltpu.sync_copy(nodes.at[pl.ds(b2, H2PW)], h2idx)
    pltpu.async_copy(features.at[h2idx], h2rows, sem0).wait()
    pltpu.sync_copy(h2rows, h2_o.at[pl.ds(b2, H2PW)])

    b1 = wid * H1PW
    pltpu.sync_copy(samples1.at[pl.ds(b1, 128 * 4)], idx_all.at[pl.ds(0, 128 * 4)])
    def h1_gather(c):
        pltpu.async_copy(
            features.at[idx_all.at[pl.ds(c * 128, 128)]],
            rows_bufs[c % 2], sems[c % 2])

    h1_gather(0)
    h1_gather(1)
    for c in range(4):
        pltpu.make_async_copy(
            features.at[idx_all.at[pl.ds(c * 128, 128)]],
            rows_bufs[c % 2], sems[c % 2]).wait()
        pltpu.sync_copy(rows_bufs[c % 2], h1_o.at[pl.ds(b1 + c * 128, 128)])
        if c + 2 < 4:
            h1_gather(c + 2)

    gbase = wid * GPW
    pltpu.sync_copy(nv0.at[pl.ds(gbase, GPW)], nv_v)
    pltpu.sync_copy(samples0.at[pl.ds(gbase * S, GPW * S)], idx_all)

    def start_gather(c, p):
        pltpu.async_copy(
            features.at[idx_all.at[pl.ds(c * CROWS, CROWS)]],
            rows_bufs[p], sems[p])

    def wait_gather(c, p):
        pltpu.make_async_copy(
            features.at[idx_all.at[pl.ds(c * CROWS, CROWS)]],
            rows_bufs[p], sems[p]).wait()

    def do_chunk(c, p, even):
        rows = rows_bufs[p]
        wait_gather(c, p)
        nvvec = nv_v[pl.ds((c // 2) * 2 * CG, 2 * CG)]
        invvec = 1.0 / nvvec.astype(jnp.float32)
        for j in range(CG):
            lbase = j * S
            nv_j = jnp.where(even, nvvec[j], nvvec[j + CG])
            inv = jnp.where(even, invvec[j], invvec[j + CG])

            def acc_row(i, accs):
                r = lbase + i
                return tuple(accs[d] + rows[r, pl.ds(d * L, L)]
                             for d in range(DCH))

            accs = lax.fori_loop(
                0, nv_j, acc_row,
                tuple(jnp.zeros((L,), jnp.float32) for _ in range(DCH)))
            g = c * CG + j
            for d in range(DCH):
                agg_v[g, pl.ds(d * L, L)] = accs[d] * inv

    start_gather(0, 0)
    start_gather(1, 1)

    def agg_pair(t, carry):
        c = 2 * t
        do_chunk(c, 0, True)

        @pl.when(c + 2 < NCH)
        def _():
            start_gather(c + 2, 0)

        do_chunk(c + 1, 1, False)

        @pl.when(c + 3 < NCH)
        def _():
            start_gather(c + 3, 1)

        return carry
    lax.fori_loop(0, NCH // 2, agg_pair, 0)

    pltpu.sync_copy(agg_v, agg0_o.at[pl.ds(gbase, GPW)])


@jax.jit
def _sc_gather(features, samples0, samples1, nodes, nv0):
    mesh = plsc.VectorSubcoreMesh(core_axis_name="c", subcore_axis_name="s",
                                  num_cores=NC, num_subcores=NS)
    return pl.kernel(
        _sc_body,
        out_type=(
            jax.ShapeDtypeStruct((B * S, D), jnp.float32),
            jax.ShapeDtypeStruct((B * S, D), jnp.float32),
            jax.ShapeDtypeStruct((B, D), jnp.float32),
        ),
        mesh=mesh,
        scratch_types=[
            pltpu.VMEM((GPW * S,), jnp.int32),
            pltpu.VMEM((CROWS, D), jnp.float32),
            pltpu.VMEM((CROWS, D), jnp.float32),
            pltpu.VMEM((GPW,), jnp.int32),
            pltpu.VMEM((GPW, D), jnp.float32),
            pltpu.VMEM((H2PW,), jnp.int32),
            pltpu.VMEM((H2PW, D), jnp.float32),
            pltpu.SemaphoreType.DMA,
            pltpu.SemaphoreType.DMA,
        ],
    )(features, samples0, samples1, nodes, nv0)


def _tc_body(h1, agg0, h2, w1, w2, m1, inv1, out):
    w1a = w1[:, :D]
    w1b = w1[:, D:]
    new1 = jnp.maximum(
        lax.dot_general(h1[:], w1a, (((1,), (1,)), ((), ())))
        + lax.dot_general(agg0[:], w1b, (((1,), (1,)), ((), ()))), 0.0)
    m = m1[:]
    agg2 = jnp.sum((new1 * m).reshape(B, S, D), axis=1) * inv1[:]
    agg1 = jnp.sum((h1[:] * m).reshape(B, S, D), axis=1) * inv1[:]
    new2 = jnp.maximum(
        lax.dot_general(h2[:], w1a, (((1,), (1,)), ((), ())))
        + lax.dot_general(agg1, w1b, (((1,), (1,)), ((), ()))), 0.0)
    out[:] = jnp.maximum(
        lax.dot_general(new2, w2[:, :D], (((1,), (1,)), ((), ())))
        + lax.dot_general(agg2, w2[:, D:], (((1,), (1,)), ((), ()))), 0.0)


@jax.jit
def _tc_combine(h1, agg0, h2, w1, w2, m1, inv1):
    return pl.pallas_call(
        _tc_body,
        out_shape=jax.ShapeDtypeStruct((B, D), jnp.float32),
    )(h1, agg0, h2, w1, w2, m1, inv1)


def kernel(features, nodes, samples1, samples0, num_valid0, num_valid1, W1, W2):
    nodes = nodes.astype(jnp.int32)
    samples1 = samples1.astype(jnp.int32)
    samples0 = samples0.astype(jnp.int32)
    nv0 = num_valid0.reshape(-1).astype(jnp.int32)
    agg0, h1, h2 = _sc_gather(features, samples0, samples1, nodes, nv0)
    m1 = (jnp.arange(S)[None, :] < num_valid1).astype(jnp.float32)
    m1 = m1.reshape(B * S, 1)
    inv1 = 1.0 / num_valid1.astype(jnp.float32)
    return _tc_combine(h1, agg0, h2, W1, W2, m1, inv1)
